# baseline (speedup 1.0000x reference)
import functools

import jax
import jax.numpy as jnp
from jax import lax
from jax.experimental import pallas as pl
from jax.experimental.pallas import tpu as pltpu
from jax.experimental.pallas import tpu_sc as plsc

_CYCLES = (7, 30, 91, 365)
_BOUNDS = tuple(c + 1 for c in _CYCLES)
_OFFS = (0, 8, 39, 131)
_ROWS = sum(_BOUNDS)
_B = 16384
_D = 64
_NC = 2
_NS = 16
_NW = _NC * _NS
_BPW = _B // _NW
_BLK = 128
_NBLK = _BPW // _BLK
_LANES = 16
_CHUNK_BYTES = _D * _BLK * 4


def _cae_body(x_hbm, tab_hbm,
              o0, o1, o2, o3, o4, o5, o6, o7,
              x_v, idx_v, tab_v, tb_a, tb_b, ssem):
    cid = lax.axis_index("c")
    sid = lax.axis_index("s")
    wid = sid * _NC + cid
    base = wid * _BPW

    pltpu.sync_copy(tab_hbm, tab_v)
    pltpu.sync_copy(x_hbm.at[pl.ds(base, _BPW)], x_v)

    def compute_idx(i, carry):
        v = x_v[pl.ds(i * _LANES, _LANES)]
        vf = v.astype(jnp.float32)
        for ci, c in enumerate(_CYCLES):
            q = (vf * jnp.float32(1.0 / c)).astype(jnp.int32)
            r = v - q * c
            r = jnp.where(r < 0, r + c, r)
            r = jnp.where(r >= c, r - c, r)
            idx_v[ci, pl.ds(i * _LANES, _LANES)] = r + (1 + _OFFS[ci])
        return carry

    lax.fori_loop(0, _BPW // _LANES, compute_idx, 0)

    iota = lax.iota(jnp.int32, _LANES)
    outs = (o0, o1, o2, o3, o4, o5, o6, o7)
    bufs = (tb_a, tb_b)

    def unit(up, carry):
        for half in range(2):
            u = up * 2 + half
            ci = u // _NBLK
            h = u % _NBLK
            tb = bufs[half]

            @pl.when(u >= 2)
            def _():
                pltpu.make_async_copy(
                    o0.at[pl.ds(0, _D), pl.ds(0, _BLK)],
                    tb.at[pl.ds(0, _D)], ssem).wait()
                pltpu.make_async_copy(
                    o0.at[pl.ds(0, _D), pl.ds(0, _BLK)],
                    tb.at[pl.ds(_D, _D)], ssem).wait()

            def fill(g, c2, tb=tb):
                r16 = idx_v[ci, pl.ds(h * _BLK + g * _LANES, _LANES)]
                b16 = g * _LANES + iota
                for c0 in range(0, 2 * _D, 16):
                    cols = [(c0 + d + iota) & (2 * _D - 1) for d in range(16)]
                    vals = [plsc.load_gather(tab_v, [r16, col])
                            for col in cols]
                    for col, val in zip(cols, vals):
                        plsc.store_scatter(tb, [col, b16], val)
                return c2

            lax.fori_loop(0, _BLK // _LANES, fill, 0)

            dst = pl.ds(base + h * _BLK, _BLK)
            for j in range(4):
                @pl.when(ci == j)
                def _(j=j, tb=tb, dst=dst):
                    pltpu.async_copy(
                        tb.at[pl.ds(0, _D)], outs[2 * j].at[:, dst], ssem)
                    pltpu.async_copy(
                        tb.at[pl.ds(_D, _D)], outs[2 * j + 1].at[:, dst],
                        ssem)
        return carry

    lax.fori_loop(0, 8, unit, 0)

    for tb in bufs:
        pltpu.make_async_copy(
            o0.at[pl.ds(0, _D), pl.ds(0, _BLK)],
            tb.at[pl.ds(0, _D)], ssem).wait()
        pltpu.make_async_copy(
            o0.at[pl.ds(0, _D), pl.ds(0, _BLK)],
            tb.at[pl.ds(_D, _D)], ssem).wait()


@jax.jit
def kernel(x, W0, W1, W2, W3, W4, W5, W6, W7):
    mesh = plsc.VectorSubcoreMesh(core_axis_name="c", subcore_axis_name="s")
    run = functools.partial(
        pl.kernel,
        mesh=mesh,
        out_type=[jax.ShapeDtypeStruct((_D, _B), jnp.float32)] * 8,
        compiler_params=pltpu.CompilerParams(needs_layout_passes=False),
        scratch_types=[
            pltpu.VMEM((_BPW,), jnp.int32),
            pltpu.VMEM((len(_CYCLES), _BPW), jnp.int32),
            pltpu.VMEM((_ROWS, 2 * _D), jnp.float32),
            pltpu.VMEM((2 * _D, _BLK), jnp.float32),
            pltpu.VMEM((2 * _D, _BLK), jnp.float32),
            pltpu.SemaphoreType.DMA,
        ],
    )(_cae_body)
    tab = jnp.concatenate(
        [jnp.concatenate([ws, wc], axis=1)
         for ws, wc in ((W0, W4), (W1, W5), (W2, W6), (W3, W7))], axis=0)
    res = run(x.astype(jnp.int32), tab)
    return tuple(r.T for r in res)


# speedup vs baseline: 1.0109x; 1.0109x over previous
import functools

import jax
import jax.numpy as jnp
from jax import lax
from jax.experimental import pallas as pl
from jax.experimental.pallas import tpu as pltpu
from jax.experimental.pallas import tpu_sc as plsc

_CYCLES = (7, 30, 91, 365)
_BOUNDS = tuple(c + 1 for c in _CYCLES)
_OFFS = (0, 8, 39, 131)
_ROWS = sum(_BOUNDS)
_B = 16384
_D = 64
_NC = 2
_NS = 16
_NW = _NC * _NS
_BPW = _B // _NW
_BLK = 128
_NBLK = _BPW // _BLK
_LANES = 16
_CHUNK_BYTES = _D * _BLK * 4


def _cae_body(x_hbm, tab_hbm,
              o0, o1, o2, o3, o4, o5, o6, o7,
              x_v, idx_v, tab_v, tb_a, tb_b, ssem):
    cid = lax.axis_index("c")
    sid = lax.axis_index("s")
    wid = sid * _NC + cid
    base = wid * _BPW

    pltpu.sync_copy(tab_hbm, tab_v)
    pltpu.sync_copy(x_hbm.at[pl.ds(base, _BPW)], x_v)

    def compute_idx(i, carry):
        v = x_v[pl.ds(i * _LANES, _LANES)]
        vf = v.astype(jnp.float32)
        for ci, c in enumerate(_CYCLES):
            q = (vf * jnp.float32(1.0 / c)).astype(jnp.int32)
            r = v - q * c
            r = jnp.where(r < 0, r + c, r)
            r = jnp.where(r >= c, r - c, r)
            idx_v[ci, pl.ds(i * _LANES, _LANES)] = r + (1 + _OFFS[ci])
        return carry

    lax.fori_loop(0, _BPW // _LANES, compute_idx, 0)

    iota = lax.iota(jnp.int32, _LANES)
    outs = (o0, o1, o2, o3, o4, o5, o6, o7)
    bufs = (tb_a, tb_b)

    def unit(up, carry):
        for half in range(2):
            u = up * 2 + half
            ci = u // _NBLK
            h = u % _NBLK
            tb = bufs[half]

            @pl.when(u >= 2)
            def _():
                pltpu.make_async_copy(
                    o0.at[pl.ds(0, _D), pl.ds(0, _BLK)],
                    tb.at[pl.ds(0, _D)], ssem).wait()
                pltpu.make_async_copy(
                    o0.at[pl.ds(0, _D), pl.ds(0, _BLK)],
                    tb.at[pl.ds(_D, _D)], ssem).wait()

            def fill(g, c2, tb=tb):
                r16 = idx_v[ci, pl.ds(h * _BLK + g * _LANES, _LANES)]
                b16 = g * _LANES + iota
                for c0 in range(0, 2 * _D, 8):
                    cols = [(c0 + d + iota) & (2 * _D - 1) for d in range(8)]
                    vals = [plsc.load_gather(tab_v, [r16, col])
                            for col in cols]
                    for col, val in zip(cols, vals):
                        plsc.store_scatter(tb, [col, b16], val)
                return c2

            lax.fori_loop(0, _BLK // _LANES, fill, 0)

            dst = pl.ds(base + h * _BLK, _BLK)
            for j in range(4):
                @pl.when(ci == j)
                def _(j=j, tb=tb, dst=dst):
                    pltpu.async_copy(
                        tb.at[pl.ds(0, _D)], outs[2 * j].at[:, dst], ssem)
                    pltpu.async_copy(
                        tb.at[pl.ds(_D, _D)], outs[2 * j + 1].at[:, dst],
                        ssem)
        return carry

    lax.fori_loop(0, 8, unit, 0)

    for tb in bufs:
        pltpu.make_async_copy(
            o0.at[pl.ds(0, _D), pl.ds(0, _BLK)],
            tb.at[pl.ds(0, _D)], ssem).wait()
        pltpu.make_async_copy(
            o0.at[pl.ds(0, _D), pl.ds(0, _BLK)],
            tb.at[pl.ds(_D, _D)], ssem).wait()


@jax.jit
def kernel(x, W0, W1, W2, W3, W4, W5, W6, W7):
    mesh = plsc.VectorSubcoreMesh(core_axis_name="c", subcore_axis_name="s")
    run = functools.partial(
        pl.kernel,
        mesh=mesh,
        out_type=[jax.ShapeDtypeStruct((_D, _B), jnp.float32)] * 8,
        compiler_params=pltpu.CompilerParams(needs_layout_passes=False),
        scratch_types=[
            pltpu.VMEM((_BPW,), jnp.int32),
            pltpu.VMEM((len(_CYCLES), _BPW), jnp.int32),
            pltpu.VMEM((_ROWS, 2 * _D), jnp.float32),
            pltpu.VMEM((2 * _D, _BLK), jnp.float32),
            pltpu.VMEM((2 * _D, _BLK), jnp.float32),
            pltpu.SemaphoreType.DMA,
        ],
    )(_cae_body)
    tab = jnp.concatenate(
        [jnp.concatenate([ws, wc], axis=1)
         for ws, wc in ((W0, W4), (W1, W5), (W2, W6), (W3, W7))], axis=0)
    res = run(x.astype(jnp.int32), tab)
    return tuple(r.T for r in res)


# speedup vs baseline: 1.0425x; 1.0312x over previous
import functools

import jax
import jax.numpy as jnp
from jax import lax
from jax.experimental import pallas as pl
from jax.experimental.pallas import tpu as pltpu
from jax.experimental.pallas import tpu_sc as plsc

_CYCLES = (7, 30, 91, 365)
_BOUNDS = tuple(c + 1 for c in _CYCLES)
_OFFS = (0, 8, 39, 131)
_ROWS = 504
_STAGE = ((0, 8), (8, 32), (40, 96), (136, 368))
_B = 16384
_D = 64
_NC = 2
_NS = 16
_NW = _NC * _NS
_BPW = _B // _NW
_BLK = 128
_NBLK = _BPW // _BLK
_LANES = 16
_CHUNK_BYTES = _D * _BLK * 4


def _cae_body(x_hbm, tab_hbm,
              o0, o1, o2, o3, o4, o5, o6, o7,
              x_v, idx_v, tab_v, tb_a, tb_b, ssem,
              ts0, ts1, ts2, ts3):
    cid = lax.axis_index("c")
    sid = lax.axis_index("s")
    wid = sid * _NC + cid
    base = wid * _BPW

    tsems = (ts0, ts1, ts2, ts3)
    for j in range(4):
        pltpu.async_copy(
            tab_hbm.at[pl.ds(_STAGE[j][0], _STAGE[j][1])],
            tab_v.at[pl.ds(_STAGE[j][0], _STAGE[j][1])], tsems[j])
    pltpu.sync_copy(x_hbm.at[pl.ds(base, _BPW)], x_v)

    def compute_idx(i, carry):
        v = x_v[pl.ds(i * _LANES, _LANES)]
        vf = v.astype(jnp.float32)
        for ci, c in enumerate(_CYCLES):
            q = (vf * jnp.float32(1.0 / c)).astype(jnp.int32)
            r = v - q * c
            r = jnp.where(r < 0, r + c, r)
            r = jnp.where(r >= c, r - c, r)
            idx_v[ci, pl.ds(i * _LANES, _LANES)] = r + (1 + _OFFS[ci])
        return carry

    lax.fori_loop(0, _BPW // _LANES, compute_idx, 0)

    iota = lax.iota(jnp.int32, _LANES)
    outs = (o0, o1, o2, o3, o4, o5, o6, o7)
    bufs = (tb_a, tb_b)

    def unit(up, carry):
        for half in range(2):
            u = up * 2 + half
            ci = u // _NBLK
            h = u % _NBLK
            tb = bufs[half]

            for j in range(4):
                @pl.when((ci == j) & (h == 0))
                def _(j=j):
                    pltpu.make_async_copy(
                        tab_hbm.at[pl.ds(_STAGE[j][0], _STAGE[j][1])],
                        tab_v.at[pl.ds(_STAGE[j][0], _STAGE[j][1])],
                        tsems[j]).wait()

            @pl.when(u >= 2)
            def _():
                pltpu.make_async_copy(
                    o0.at[pl.ds(0, _D), pl.ds(0, _BLK)],
                    tb.at[pl.ds(0, _D)], ssem).wait()
                pltpu.make_async_copy(
                    o0.at[pl.ds(0, _D), pl.ds(0, _BLK)],
                    tb.at[pl.ds(_D, _D)], ssem).wait()

            def fill(g, c2, tb=tb):
                r16 = idx_v[ci, pl.ds(h * _BLK + g * _LANES, _LANES)]
                b16 = g * _LANES + iota
                for c0 in range(0, 2 * _D, 8):
                    cols = [(c0 + d + iota) & (2 * _D - 1) for d in range(8)]
                    vals = [plsc.load_gather(tab_v, [r16, col])
                            for col in cols]
                    for col, val in zip(cols, vals):
                        plsc.store_scatter(tb, [col, b16], val)
                return c2

            lax.fori_loop(0, _BLK // _LANES, fill, 0)

            dst = pl.ds(base + h * _BLK, _BLK)
            for j in range(4):
                @pl.when(ci == j)
                def _(j=j, tb=tb, dst=dst):
                    pltpu.async_copy(
                        tb.at[pl.ds(0, _D)], outs[2 * j].at[:, dst], ssem)
                    pltpu.async_copy(
                        tb.at[pl.ds(_D, _D)], outs[2 * j + 1].at[:, dst],
                        ssem)
        return carry

    lax.fori_loop(0, 8, unit, 0)

    for tb in bufs:
        pltpu.make_async_copy(
            o0.at[pl.ds(0, _D), pl.ds(0, _BLK)],
            tb.at[pl.ds(0, _D)], ssem).wait()
        pltpu.make_async_copy(
            o0.at[pl.ds(0, _D), pl.ds(0, _BLK)],
            tb.at[pl.ds(_D, _D)], ssem).wait()


@jax.jit
def kernel(x, W0, W1, W2, W3, W4, W5, W6, W7):
    mesh = plsc.VectorSubcoreMesh(core_axis_name="c", subcore_axis_name="s")
    run = functools.partial(
        pl.kernel,
        mesh=mesh,
        out_type=[jax.ShapeDtypeStruct((_D, _B), jnp.float32)] * 8,
        compiler_params=pltpu.CompilerParams(needs_layout_passes=False),
        scratch_types=[
            pltpu.VMEM((_BPW,), jnp.int32),
            pltpu.VMEM((len(_CYCLES), _BPW), jnp.int32),
            pltpu.VMEM((_ROWS, 2 * _D), jnp.float32),
            pltpu.VMEM((2 * _D, _BLK), jnp.float32),
            pltpu.VMEM((2 * _D, _BLK), jnp.float32),
            pltpu.SemaphoreType.DMA,
            pltpu.SemaphoreType.DMA,
            pltpu.SemaphoreType.DMA,
            pltpu.SemaphoreType.DMA,
            pltpu.SemaphoreType.DMA,
        ],
    )(_cae_body)
    tab = jnp.concatenate(
        [jnp.concatenate([ws, wc], axis=1)
         for ws, wc in ((W0, W4), (W1, W5), (W2, W6), (W3, W7))]
        + [jnp.zeros((_ROWS - 497, 2 * _D), jnp.float32)], axis=0)
    res = run(x.astype(jnp.int32), tab)
    return tuple(r.T for r in res)
